# Initial kernel scaffold; baseline (speedup 1.0000x reference)
import jax
import jax.numpy as jnp
from jax.experimental import pallas as pl


def kernel(y_pred, y_true):
    raise NotImplementedError("write your pallas kernel here")


# speedup vs baseline: 8.0069x; 8.0069x over previous
import functools

import jax
import jax.numpy as jnp
from jax import lax
from jax.experimental import pallas as pl
from jax.experimental.pallas import tpu as pltpu

_IGNORE = -1
_THRESH_BITS = 0x3F333333
_MIN_KEPT = 100000

_B, _C, _H, _W = 4, 150, 512, 512
_HB = 16


def _stage1_body(yt_ref, yp_ref, ce_ref, p_ref):
    x = yp_ref[0]
    lbl = yt_ref[0]
    m = jnp.max(x, axis=0)
    e = jnp.exp(x - m[None, :, :])
    s = jnp.sum(e, axis=0)
    cls = lax.broadcasted_iota(jnp.int32, x.shape, 0)
    xt = jnp.sum(jnp.where(cls == lbl[None, :, :], x, 0.0), axis=0)
    ce = (m - xt) + jnp.log(s)
    ce_ref[0] = ce
    p_ref[0] = jnp.exp(xt - m) / s


def _stage2_body(batch_kept, p_ref, ce_ref, out_ref):
    ip = lax.bitcast_convert_type(p_ref[...], jnp.int32)
    k1 = batch_kept + 1

    def bs_body(_, lohi):
        lo, hi = lohi
        mid = lo + (hi - lo) // 2
        cnt = jnp.sum((ip <= mid).astype(jnp.int32))
        take = cnt >= k1
        return (jnp.where(take, lo, mid + 1), jnp.where(take, mid, hi))

    _, kth = lax.fori_loop(0, 31, bs_body, (jnp.int32(0), jnp.int32(0x7F000000)))
    thr_bits = jnp.maximum(kth, jnp.int32(_THRESH_BITS))
    w = (ip < thr_bits).astype(jnp.float32)
    num = jnp.sum(ce_ref[...] * w)
    den = jnp.sum(w)
    out_ref[0, 0] = num / den


@jax.jit
def kernel(y_pred, y_true):
    b, c, h, w = y_pred.shape
    grid = (b, h // _HB)
    ce, p = pl.pallas_call(
        _stage1_body,
        grid=grid,
        in_specs=[
            pl.BlockSpec((1, _HB, w), lambda i, j: (i, j, 0)),
            pl.BlockSpec((1, c, _HB, w), lambda i, j: (i, 0, j, 0)),
        ],
        out_specs=[
            pl.BlockSpec((1, _HB, w), lambda i, j: (i, j, 0)),
            pl.BlockSpec((1, _HB, w), lambda i, j: (i, j, 0)),
        ],
        out_shape=[
            jax.ShapeDtypeStruct((b, h, w), jnp.float32),
            jax.ShapeDtypeStruct((b, h, w), jnp.float32),
        ],
    )(y_true, y_pred)

    out = pl.pallas_call(
        functools.partial(_stage2_body, _MIN_KEPT * b),
        out_shape=jax.ShapeDtypeStruct((1, 1), jnp.float32),
        out_specs=pl.BlockSpec(memory_space=pltpu.SMEM),
    )(p, ce)
    return out[0, 0]
